# baseline (speedup 1.0000x reference)
import functools

import jax
import jax.numpy as jnp
from jax import lax
from jax.experimental import pallas as pl
from jax.experimental.pallas import tpu as pltpu
from jax.experimental.pallas import tpu_sc as plsc

N = 10000
C = 256
H = 8
HD = 32
HID = 1024

NW = 32
NB = 313
NB1 = NB + 1
NPAD = NW * NB
L = 16
EB = 48
SUP = 1536
NBLK = SUP // EB
E_RAW = 2 * 160000 + 20000
E_PAD = E_RAW + SUP
BIG = 1 << 20


def _ln(x, g, b):
    mu = jnp.mean(x, axis=-1, keepdims=True)
    var = jnp.mean((x - mu) ** 2, axis=-1, keepdims=True)
    return (x - mu) / jnp.sqrt(var + 1e-5) * g + b


def _qkv_body(f_ref, g_ref, b_ref, w_ref, wb_ref, q_ref, k_ref, v_ref):
    x = _ln(f_ref[...], g_ref[...], b_ref[...])
    qkv = lax.dot_general(x, w_ref[...], (((1,), (1,)), ((), ())),
                          preferred_element_type=jnp.float32) + wb_ref[...]
    q_ref[...] = qkv[:, :C] * (HD ** -0.5)
    k_ref[...] = qkv[:, C:2 * C]
    v_ref[...] = qkv[:, 2 * C:]


def _qkv_tc(feats, g, b, w, wb):
    R = 400
    grid = (N // R,)
    return pl.pallas_call(
        _qkv_body,
        grid=grid,
        in_specs=[
            pl.BlockSpec((R, C), lambda i: (i, 0)),
            pl.BlockSpec((C,), lambda i: (0,)),
            pl.BlockSpec((C,), lambda i: (0,)),
            pl.BlockSpec((3 * C, C), lambda i: (0, 0)),
            pl.BlockSpec((3 * C,), lambda i: (0,)),
        ],
        out_specs=[
            pl.BlockSpec((R, C), lambda i: (i, 0)),
            pl.BlockSpec((R, C), lambda i: (i, 0)),
            pl.BlockSpec((R, C), lambda i: (i, 0)),
        ],
        out_shape=[jax.ShapeDtypeStruct((N, C), jnp.float32)] * 3,
    )(feats, g, b, w, wb)


def _tail_body(ao_ref, f_ref, pw_ref, pb_ref, g2_ref, b2_ref,
               w1_ref, b1_ref, w2_ref, b2b_ref, o_ref):
    y = lax.dot_general(ao_ref[...], pw_ref[...], (((1,), (1,)), ((), ())),
                        preferred_element_type=jnp.float32) + pb_ref[...]
    f2 = f_ref[...] + y
    h = _ln(f2, g2_ref[...], b2_ref[...])
    h = lax.dot_general(h, w1_ref[...], (((1,), (1,)), ((), ())),
                        preferred_element_type=jnp.float32) + b1_ref[...]
    h = 0.5 * h * (1.0 + lax.erf(h * (2.0 ** -0.5)))
    h = lax.dot_general(h, w2_ref[...], (((1,), (1,)), ((), ())),
                        preferred_element_type=jnp.float32) + b2b_ref[...]
    o_ref[...] = f2 + h


def _tail_tc(attn_out, feats, pw, pb, g2, b2, w1, b1, w2, b2b):
    R = 400
    grid = (N // R,)
    return pl.pallas_call(
        _tail_body,
        grid=grid,
        in_specs=[
            pl.BlockSpec((R, C), lambda i: (i, 0)),
            pl.BlockSpec((R, C), lambda i: (i, 0)),
            pl.BlockSpec((C, C), lambda i: (0, 0)),
            pl.BlockSpec((C,), lambda i: (0,)),
            pl.BlockSpec((C,), lambda i: (0,)),
            pl.BlockSpec((C,), lambda i: (0,)),
            pl.BlockSpec((HID, C), lambda i: (0, 0)),
            pl.BlockSpec((HID,), lambda i: (0,)),
            pl.BlockSpec((C, HID), lambda i: (0, 0)),
            pl.BlockSpec((C,), lambda i: (0,)),
        ],
        out_specs=pl.BlockSpec((R, C), lambda i: (i, 0)),
        out_shape=jax.ShapeDtypeStruct((N, C), jnp.float32),
    )(attn_out, feats, pw, pb, g2, b2, w1, b1, w2, b2b)


def _sc_attention(i0s, i1, q, k, v, ts):
    mesh = plsc.VectorSubcoreMesh(core_axis_name="c", subcore_axis_name="s")

    @functools.partial(
        pl.kernel,
        mesh=mesh,
        compiler_params=pltpu.CompilerParams(needs_layout_passes=False),
        out_type=[
            jax.ShapeDtypeStruct((NPAD * C,), jnp.float32),
            jax.ShapeDtypeStruct((E_PAD * 8,), jnp.float32),
        ],
        scratch_types=[
            pltpu.VMEM((48,), jnp.int32),
            pltpu.VMEM((SUP,), jnp.int32),
            pltpu.VMEM((SUP,), jnp.int32),
            pltpu.VMEM((EB, C), jnp.float32),
            pltpu.VMEM((EB, C), jnp.float32),
            pltpu.VMEM((SUP * 8,), jnp.float32),
            pltpu.VMEM((NB1 * C,), jnp.float32),
            pltpu.VMEM((NB1 * 8,), jnp.float32),
            pltpu.SemaphoreType.DMA,
            pltpu.SemaphoreType.DMA,
        ],
    )
    def run(i0s_hbm, i1_hbm, q_hbm, k_hbm, v_hbm, ts_hbm,
            out_hbm, attn_hbm,
            ts_v, i0_sup, i1_sup, rows0, rows1, attn_sup, qo_buf,
            segsum_v, sem0, sem1):
        cid = lax.axis_index("c")
        sid = lax.axis_index("s")
        wid = cid * 16 + sid
        nodebase = wid * NB

        iota = lax.iota(jnp.int32, L)
        zero16 = jnp.zeros((L,), jnp.float32)

        pltpu.sync_copy(ts_hbm, ts_v)
        tsvec = plsc.load_gather(ts_v, [jnp.full((L,), wid, jnp.int32) + iota])
        start = tsvec[0]
        end = tsvec[1]
        base8 = (start // 8) * 8
        nsup = (end - base8 + SUP - 1) // SUP

        pltpu.sync_copy(q_hbm.at[pl.ds(nodebase * C, NB * C)],
                        qo_buf.at[pl.ds(0, NB * C)])

        rows = (rows0, rows1)
        sems = (sem0, sem1)

        def wait_rows(slot):
            pltpu.make_async_copy(k_hbm.at[pl.ds(0, EB)], rows[slot],
                                  sems[slot]).wait()

        masks = [iota == i for i in range(L)]

        def pass_a(s, maxc):
            sup_off = base8 + s * SUP
            pltpu.sync_copy(i1_hbm.at[pl.ds(sup_off, SUP)], i1_sup)
            pltpu.sync_copy(i0s_hbm.at[pl.ds(sup_off, SUP)], i0_sup)
            pltpu.async_copy(k_hbm.at[i1_sup.at[pl.ds(0, EB)]], rows0, sem0)

            def compute_blk(j, slot, mc):
                def grp(g, mcv):
                    base_e = j * EB + g * L
                    iv16 = i0_sup[pl.ds(base_e, L)]
                    for te in range(0, L, 2):
                        av = zero16
                        for t in (te, te + 1):
                            i0l = jnp.clip(iv16[t] - nodebase, 0, NB - 1)
                            qoff = i0l * C
                            erow_t = g * L + t
                            ps = []
                            for c in range(C // L):
                                qc = qo_buf[pl.ds(qoff + c * L, L)]
                                kc = rows[slot][erow_t, pl.ds(c * L, L)]
                                ps.append(qc * kc)
                            for h in range(H):
                                sh = ps[2 * h] + ps[2 * h + 1]
                                tot = plsc.cumsum(sh)[L - 1]
                                av = jnp.where(masks[h + 8 * (t - te)],
                                               tot, av)
                        attn_sup[pl.ds((base_e + te) * 8, L)] = av
                        mcv = jnp.maximum(mcv, av)
                    return mcv

                return lax.fori_loop(0, EB // L, grp, mc)

            def pair(jj, mc):
                j0 = 2 * jj
                cp1 = pltpu.async_copy(
                    k_hbm.at[i1_sup.at[pl.ds((j0 + 1) * EB, EB)]], rows1, sem1)
                wait_rows(0)
                mc = compute_blk(j0, 0, mc)

                @pl.when(jj < NBLK // 2 - 1)
                def _():
                    pltpu.async_copy(
                        k_hbm.at[i1_sup.at[pl.ds((j0 + 2) * EB, EB)]],
                        rows0, sem0)

                cp1.wait()
                mc = compute_blk(j0 + 1, 1, mc)
                return mc

            mc = lax.fori_loop(0, NBLK // 2, pair, maxc)
            pltpu.sync_copy(attn_sup, attn_hbm.at[pl.ds(sup_off * 8, SUP * 8)])
            return mc

        minf = jnp.full((L,), -1e30, jnp.float32)
        maxc = lax.fori_loop(0, nsup, pass_a, minf)

        kv16 = zero16
        for h in range(H):
            kh = jnp.maximum(maxc[h], maxc[h + 8])
            kv16 = jnp.where((iota % 8) == h, kh, kv16)

        def _z1(i, _):
            segsum_v[pl.ds(i * L, L)] = zero16
            return 0
        lax.fori_loop(0, NB1 * 8 // L, _z1, 0)

        lo8 = iota < 8
        hmod = iota % 8

        def pair_seg(iv16, te, epos0):
            sis = []
            oks = []
            for t in (te, te + 1):
                i0l = iv16[t] - nodebase
                ep = epos0 + t
                ok = ((i0l >= 0) & (i0l < NB) & (ep >= start) & (ep < end))
                sis.append(jnp.where(ok, i0l, NB))
                oks.append(ok)
            sidx = jnp.where(lo8, sis[0] * 8, sis[1] * 8) + hmod
            okv = jnp.where(lo8, oks[0], oks[1])
            return sidx, okv

        def pass_b(s, _):
            sup_off = base8 + s * SUP
            pltpu.sync_copy(i0s_hbm.at[pl.ds(sup_off, SUP)], i0_sup)
            pltpu.sync_copy(attn_hbm.at[pl.ds(sup_off * 8, SUP * 8)], attn_sup)

            def grp(g, _):
                base_e = g * L
                iv16 = i0_sup[pl.ds(base_e, L)]
                epos0 = sup_off + base_e
                for te in range(0, L, 2):
                    a = attn_sup[pl.ds((base_e + te) * 8, L)]
                    e = jnp.exp(a - kv16)
                    sidx, okv = pair_seg(iv16, te, epos0)
                    e = jnp.where(okv, e, 0.0)
                    plsc.addupdate_scatter(segsum_v, [sidx], e)
                return 0

            lax.fori_loop(0, SUP // L, grp, 0)
            return 0

        lax.fori_loop(0, nsup, pass_b, 0)

        def _inv(i, _):
            seg = segsum_v[pl.ds(i * L, L)]
            segsum_v[pl.ds(i * L, L)] = 1.0 / seg
            return 0
        lax.fori_loop(0, NB1 * 8 // L, _inv, 0)

        def _z2(i, _):
            qo_buf[pl.ds(i * L, L)] = zero16
            return 0
        lax.fori_loop(0, NB1 * C // L, _z2, 0)

        def pass_c(s, _):
            sup_off = base8 + s * SUP
            pltpu.sync_copy(i1_hbm.at[pl.ds(sup_off, SUP)], i1_sup)
            pltpu.sync_copy(i0s_hbm.at[pl.ds(sup_off, SUP)], i0_sup)
            pltpu.sync_copy(attn_hbm.at[pl.ds(sup_off * 8, SUP * 8)], attn_sup)
            pltpu.async_copy(v_hbm.at[i1_sup.at[pl.ds(0, EB)]], rows0, sem0)

            def compute_blk(j, slot, _):
                def grp(g, _):
                    base_e = j * EB + g * L
                    iv16 = i0_sup[pl.ds(base_e, L)]
                    epos0 = sup_off + base_e
                    for te in range(0, L, 2):
                        a = attn_sup[pl.ds((base_e + te) * 8, L)]
                        ex = jnp.exp(a - kv16)
                        sidx, okv = pair_seg(iv16, te, epos0)
                        recip = plsc.load_gather(segsum_v, [sidx])
                        s16 = jnp.where(okv, ex * recip, 0.0)
                        for t in (te, te + 1):
                            i0l = iv16[t] - nodebase
                            si = jnp.where((i0l >= 0) & (i0l < NB), i0l, NB)
                            erow_t = g * L + t
                            off = 8 * (t - te)
                            for c in range(C // L):
                                sc = s16[c // 2 + off]
                                chunk = rows[slot][erow_t,
                                                   pl.ds(c * L, L)] * sc
                                plsc.addupdate(
                                    qo_buf.at[pl.ds(si * C + c * L, L)],
                                    chunk)
                    return 0

                lax.fori_loop(0, EB // L, grp, 0)
                return 0

            def pair(jj, _):
                j0 = 2 * jj
                cp1 = pltpu.async_copy(
                    v_hbm.at[i1_sup.at[pl.ds((j0 + 1) * EB, EB)]], rows1, sem1)
                wait_rows(0)
                compute_blk(j0, 0, 0)

                @pl.when(jj < NBLK // 2 - 1)
                def _():
                    pltpu.async_copy(
                        v_hbm.at[i1_sup.at[pl.ds((j0 + 2) * EB, EB)]],
                        rows0, sem0)

                cp1.wait()
                compute_blk(j0 + 1, 1, 0)
                return 0

            lax.fori_loop(0, NBLK // 2, pair, 0)
            return 0

        lax.fori_loop(0, nsup, pass_c, 0)

        pltpu.sync_copy(qo_buf.at[pl.ds(0, NB * C)],
                        out_hbm.at[pl.ds(nodebase * C, NB * C)])

    return run(i0s, i1, q, k, v, ts)


def kernel(feats, xyz, temporal_edge_index, spatial_edge_index, batch,
           norm1_g, norm1_b, qkv_w, qkv_b, proj_w, proj_b,
           norm2_g, norm2_b, fc1_w, fc1_b, fc2_w, fc2_b):
    n = feats.shape[0]

    ei = jnp.concatenate([spatial_edge_index, spatial_edge_index[::-1, :],
                          temporal_edge_index], axis=1)
    keys = ei[0] * n + ei[1]
    sk = jnp.sort(keys)
    valid = jnp.concatenate([jnp.ones((1,), dtype=bool), sk[1:] != sk[:-1]])
    i0_all = sk // n
    i1 = jnp.where(valid, sk % n, 0).astype(jnp.int32)
    i0s = jnp.where(valid, i0_all, BIG).astype(jnp.int32)

    pad = E_PAD - E_RAW
    i0s = jnp.concatenate([i0s, jnp.full((pad,), BIG, jnp.int32)])
    i1 = jnp.concatenate([i1, jnp.zeros((pad,), jnp.int32)])

    bounds = (jnp.arange(33, dtype=jnp.int32) * NB).astype(jnp.int32)
    ts = jnp.searchsorted(i0_all.astype(jnp.int32), bounds).astype(jnp.int32)
    ts = jnp.concatenate([ts, jnp.zeros((15,), jnp.int32)])

    q, k, v = _qkv_tc(feats, norm1_g, norm1_b, qkv_w, qkv_b)
    q = jnp.pad(q, ((0, NPAD - N), (0, 0))).reshape(NPAD * C)

    out_pad, _ = _sc_attention(i0s, i1, q, k, v, ts)
    attn_out = out_pad.reshape(NPAD, C)[:n]

    return _tail_tc(attn_out, feats, proj_w, proj_b, norm2_g, norm2_b,
                    fc1_w, fc1_b, fc2_w, fc2_b)


# speedup vs baseline: 1.0089x; 1.0089x over previous
import functools

import jax
import jax.numpy as jnp
from jax import lax
from jax.experimental import pallas as pl
from jax.experimental.pallas import tpu as pltpu
from jax.experimental.pallas import tpu_sc as plsc

N = 10000
C = 256
H = 8
HD = 32
HID = 1024

NW = 32
NB = 313
NB1 = NB + 1
NPAD = NW * NB
L = 16
EB = 48
SUP = 1536
NBLK = SUP // EB
E_RAW = 2 * 160000 + 20000
E_PAD = E_RAW + SUP
BIG = 1 << 20


def _ln(x, g, b):
    mu = jnp.mean(x, axis=-1, keepdims=True)
    var = jnp.mean((x - mu) ** 2, axis=-1, keepdims=True)
    return (x - mu) / jnp.sqrt(var + 1e-5) * g + b


def _qkv_body(f_ref, g_ref, b_ref, w_ref, wb_ref, q_ref, k_ref, v_ref):
    x = _ln(f_ref[...], g_ref[...], b_ref[...])
    qkv = lax.dot_general(x, w_ref[...], (((1,), (1,)), ((), ())),
                          preferred_element_type=jnp.float32) + wb_ref[...]
    q_ref[...] = qkv[:, :C] * (HD ** -0.5)
    k_ref[...] = qkv[:, C:2 * C]
    v_ref[...] = qkv[:, 2 * C:]


def _qkv_tc(feats, g, b, w, wb):
    R = 400
    grid = (N // R,)
    return pl.pallas_call(
        _qkv_body,
        grid=grid,
        in_specs=[
            pl.BlockSpec((R, C), lambda i: (i, 0)),
            pl.BlockSpec((C,), lambda i: (0,)),
            pl.BlockSpec((C,), lambda i: (0,)),
            pl.BlockSpec((3 * C, C), lambda i: (0, 0)),
            pl.BlockSpec((3 * C,), lambda i: (0,)),
        ],
        out_specs=[
            pl.BlockSpec((R, C), lambda i: (i, 0)),
            pl.BlockSpec((R, C), lambda i: (i, 0)),
            pl.BlockSpec((R, C), lambda i: (i, 0)),
        ],
        out_shape=[jax.ShapeDtypeStruct((N, C), jnp.float32)] * 3,
    )(feats, g, b, w, wb)


def _tail_body(ao_ref, f_ref, pw_ref, pb_ref, g2_ref, b2_ref,
               w1_ref, b1_ref, w2_ref, b2b_ref, o_ref):
    y = lax.dot_general(ao_ref[...], pw_ref[...], (((1,), (1,)), ((), ())),
                        preferred_element_type=jnp.float32) + pb_ref[...]
    f2 = f_ref[...] + y
    h = _ln(f2, g2_ref[...], b2_ref[...])
    h = lax.dot_general(h, w1_ref[...], (((1,), (1,)), ((), ())),
                        preferred_element_type=jnp.float32) + b1_ref[...]
    h = 0.5 * h * (1.0 + lax.erf(h * (2.0 ** -0.5)))
    h = lax.dot_general(h, w2_ref[...], (((1,), (1,)), ((), ())),
                        preferred_element_type=jnp.float32) + b2b_ref[...]
    o_ref[...] = f2 + h


def _tail_tc(attn_out, feats, pw, pb, g2, b2, w1, b1, w2, b2b):
    R = 400
    grid = (N // R,)
    return pl.pallas_call(
        _tail_body,
        grid=grid,
        in_specs=[
            pl.BlockSpec((R, C), lambda i: (i, 0)),
            pl.BlockSpec((R, C), lambda i: (i, 0)),
            pl.BlockSpec((C, C), lambda i: (0, 0)),
            pl.BlockSpec((C,), lambda i: (0,)),
            pl.BlockSpec((C,), lambda i: (0,)),
            pl.BlockSpec((C,), lambda i: (0,)),
            pl.BlockSpec((HID, C), lambda i: (0, 0)),
            pl.BlockSpec((HID,), lambda i: (0,)),
            pl.BlockSpec((C, HID), lambda i: (0, 0)),
            pl.BlockSpec((C,), lambda i: (0,)),
        ],
        out_specs=pl.BlockSpec((R, C), lambda i: (i, 0)),
        out_shape=jax.ShapeDtypeStruct((N, C), jnp.float32),
    )(attn_out, feats, pw, pb, g2, b2, w1, b1, w2, b2b)


def _sc_attention(i0s, i1, q, k, v, ts):
    mesh = plsc.VectorSubcoreMesh(core_axis_name="c", subcore_axis_name="s")

    @functools.partial(
        pl.kernel,
        mesh=mesh,
        compiler_params=pltpu.CompilerParams(needs_layout_passes=False),
        out_type=[
            jax.ShapeDtypeStruct((NPAD * C,), jnp.float32),
            jax.ShapeDtypeStruct((E_PAD * 8,), jnp.float32),
        ],
        scratch_types=[
            pltpu.VMEM((48,), jnp.int32),
            pltpu.VMEM((SUP,), jnp.int32),
            pltpu.VMEM((SUP,), jnp.int32),
            pltpu.VMEM((EB, C), jnp.float32),
            pltpu.VMEM((EB, C), jnp.float32),
            pltpu.VMEM((SUP * 8,), jnp.float32),
            pltpu.VMEM((NB1 * C,), jnp.float32),
            pltpu.VMEM((NB1 * 8,), jnp.float32),
            pltpu.SemaphoreType.DMA,
            pltpu.SemaphoreType.DMA,
        ],
    )
    def run(i0s_hbm, i1_hbm, q_hbm, k_hbm, v_hbm, ts_hbm,
            out_hbm, attn_hbm,
            ts_v, i0_sup, i1_sup, rows0, rows1, attn_sup, qo_buf,
            segsum_v, sem0, sem1):
        cid = lax.axis_index("c")
        sid = lax.axis_index("s")
        wid = cid * 16 + sid
        nodebase = wid * NB

        iota = lax.iota(jnp.int32, L)
        zero16 = jnp.zeros((L,), jnp.float32)

        pltpu.sync_copy(ts_hbm, ts_v)
        tsvec = plsc.load_gather(ts_v, [jnp.full((L,), wid, jnp.int32) + iota])
        start = tsvec[0]
        end = tsvec[1]
        base8 = (start // 8) * 8
        nsup = (end - base8 + SUP - 1) // SUP

        pltpu.sync_copy(q_hbm.at[pl.ds(nodebase * C, NB * C)],
                        qo_buf.at[pl.ds(0, NB * C)])

        rows = (rows0, rows1)
        sems = (sem0, sem1)

        def wait_rows(slot):
            pltpu.make_async_copy(k_hbm.at[pl.ds(0, EB)], rows[slot],
                                  sems[slot]).wait()

        masks = [iota == i for i in range(L)]

        def pass_a(s, maxc):
            sup_off = base8 + s * SUP
            pltpu.sync_copy(i1_hbm.at[pl.ds(sup_off, SUP)], i1_sup)
            pltpu.sync_copy(i0s_hbm.at[pl.ds(sup_off, SUP)], i0_sup)
            pltpu.async_copy(k_hbm.at[i1_sup.at[pl.ds(0, EB)]], rows0, sem0)

            def compute_blk(j, slot, mc):
                def grp(g, mcv):
                    base_e = j * EB + g * L
                    iv16 = i0_sup[pl.ds(base_e, L)]
                    for te in range(0, L, 2):
                        av = zero16
                        for t in (te, te + 1):
                            i0l = jnp.clip(iv16[t] - nodebase, 0, NB - 1)
                            qoff = i0l * C
                            erow_t = g * L + t
                            ps = []
                            for c in range(C // L):
                                qc = qo_buf[pl.ds(qoff + c * L, L)]
                                kc = rows[slot][erow_t, pl.ds(c * L, L)]
                                ps.append(qc * kc)
                            for h in range(H):
                                sh = ps[2 * h] + ps[2 * h + 1]
                                tot = plsc.cumsum(sh)[L - 1]
                                av = jnp.where(masks[h + 8 * (t - te)],
                                               tot, av)
                        attn_sup[pl.ds((base_e + te) * 8, L)] = av
                        mcv = jnp.maximum(mcv, av)
                    return mcv

                return lax.fori_loop(0, EB // L, grp, mc)

            def pair(jj, mc):
                j0 = 2 * jj
                cp1 = pltpu.async_copy(
                    k_hbm.at[i1_sup.at[pl.ds((j0 + 1) * EB, EB)]], rows1, sem1)
                wait_rows(0)
                mc = compute_blk(j0, 0, mc)

                @pl.when(jj < NBLK // 2 - 1)
                def _():
                    pltpu.async_copy(
                        k_hbm.at[i1_sup.at[pl.ds((j0 + 2) * EB, EB)]],
                        rows0, sem0)

                cp1.wait()
                mc = compute_blk(j0 + 1, 1, mc)
                return mc

            mc = lax.fori_loop(0, NBLK // 2, pair, maxc)
            pltpu.sync_copy(attn_sup, attn_hbm.at[pl.ds(sup_off * 8, SUP * 8)])
            return mc

        minf = jnp.full((L,), -1e30, jnp.float32)
        maxc = lax.fori_loop(0, nsup, pass_a, minf)

        kv16 = zero16
        for h in range(H):
            kh = jnp.maximum(maxc[h], maxc[h + 8])
            kv16 = jnp.where((iota % 8) == h, kh, kv16)

        def _z1(i, _):
            segsum_v[pl.ds(i * L, L)] = zero16
            return 0
        lax.fori_loop(0, NB1 * 8 // L, _z1, 0)

        lo8 = iota < 8
        hmod = iota % 8

        def pair_seg(iv16, te, epos0):
            sis = []
            oks = []
            for t in (te, te + 1):
                i0l = iv16[t] - nodebase
                ep = epos0 + t
                ok = ((i0l >= 0) & (i0l < NB) & (ep >= start) & (ep < end))
                sis.append(jnp.where(ok, i0l, NB))
                oks.append(ok)
            sidx = jnp.where(lo8, sis[0] * 8, sis[1] * 8) + hmod
            okv = jnp.where(lo8, oks[0], oks[1])
            return sidx, okv

        def pass_b(s, _):
            sup_off = base8 + s * SUP
            pltpu.sync_copy(i0s_hbm.at[pl.ds(sup_off, SUP)], i0_sup)
            pltpu.sync_copy(attn_hbm.at[pl.ds(sup_off * 8, SUP * 8)], attn_sup)

            def grp(g, _):
                base_e = g * L
                iv16 = i0_sup[pl.ds(base_e, L)]
                epos0 = sup_off + base_e
                for te in range(0, L, 2):
                    a = attn_sup[pl.ds((base_e + te) * 8, L)]
                    e = jnp.exp(a - kv16)
                    sidx, okv = pair_seg(iv16, te, epos0)
                    e = jnp.where(okv, e, 0.0)
                    plsc.addupdate_scatter(segsum_v, [sidx], e)
                return 0

            lax.fori_loop(0, SUP // L, grp, 0)
            return 0

        lax.fori_loop(0, nsup, pass_b, 0)

        def _z2(i, _):
            qo_buf[pl.ds(i * L, L)] = zero16
            return 0
        lax.fori_loop(0, NB1 * C // L, _z2, 0)

        def pass_c(s, _):
            sup_off = base8 + s * SUP
            pltpu.sync_copy(i1_hbm.at[pl.ds(sup_off, SUP)], i1_sup)
            pltpu.sync_copy(i0s_hbm.at[pl.ds(sup_off, SUP)], i0_sup)
            pltpu.sync_copy(attn_hbm.at[pl.ds(sup_off * 8, SUP * 8)], attn_sup)
            pltpu.async_copy(v_hbm.at[i1_sup.at[pl.ds(0, EB)]], rows0, sem0)

            def compute_blk(j, slot, _):
                def grp(g, _):
                    base_e = j * EB + g * L
                    iv16 = i0_sup[pl.ds(base_e, L)]
                    epos0 = sup_off + base_e
                    for te in range(0, L, 2):
                        a = attn_sup[pl.ds((base_e + te) * 8, L)]
                        ex = jnp.exp(a - kv16)
                        sidx, okv = pair_seg(iv16, te, epos0)
                        denom = plsc.load_gather(segsum_v, [sidx])
                        s16 = jnp.where(okv, ex / denom, 0.0)
                        for t in (te, te + 1):
                            i0l = iv16[t] - nodebase
                            si = jnp.where((i0l >= 0) & (i0l < NB), i0l, NB)
                            erow_t = g * L + t
                            off = 8 * (t - te)
                            for c in range(C // L):
                                sc = s16[c // 2 + off]
                                chunk = rows[slot][erow_t,
                                                   pl.ds(c * L, L)] * sc
                                plsc.addupdate(
                                    qo_buf.at[pl.ds(si * C + c * L, L)],
                                    chunk)
                    return 0

                lax.fori_loop(0, EB // L, grp, 0)
                return 0

            def pair(jj, _):
                j0 = 2 * jj
                cp1 = pltpu.async_copy(
                    v_hbm.at[i1_sup.at[pl.ds((j0 + 1) * EB, EB)]], rows1, sem1)
                wait_rows(0)
                compute_blk(j0, 0, 0)

                @pl.when(jj < NBLK // 2 - 1)
                def _():
                    pltpu.async_copy(
                        v_hbm.at[i1_sup.at[pl.ds((j0 + 2) * EB, EB)]],
                        rows0, sem0)

                cp1.wait()
                compute_blk(j0 + 1, 1, 0)
                return 0

            lax.fori_loop(0, NBLK // 2, pair, 0)
            return 0

        lax.fori_loop(0, nsup, pass_c, 0)

        pltpu.sync_copy(qo_buf.at[pl.ds(0, NB * C)],
                        out_hbm.at[pl.ds(nodebase * C, NB * C)])

    return run(i0s, i1, q, k, v, ts)


def kernel(feats, xyz, temporal_edge_index, spatial_edge_index, batch,
           norm1_g, norm1_b, qkv_w, qkv_b, proj_w, proj_b,
           norm2_g, norm2_b, fc1_w, fc1_b, fc2_w, fc2_b):
    n = feats.shape[0]

    ei = jnp.concatenate([spatial_edge_index, spatial_edge_index[::-1, :],
                          temporal_edge_index], axis=1)
    keys = ei[0] * n + ei[1]
    sk = jnp.sort(keys)
    valid = jnp.concatenate([jnp.ones((1,), dtype=bool), sk[1:] != sk[:-1]])
    i0_all = sk // n
    i1 = jnp.where(valid, sk % n, 0).astype(jnp.int32)
    i0s = jnp.where(valid, i0_all, BIG).astype(jnp.int32)

    pad = E_PAD - E_RAW
    i0s = jnp.concatenate([i0s, jnp.full((pad,), BIG, jnp.int32)])
    i1 = jnp.concatenate([i1, jnp.zeros((pad,), jnp.int32)])

    bounds = (jnp.arange(33, dtype=jnp.int32) * NB).astype(jnp.int32)
    ts = jnp.searchsorted(i0_all.astype(jnp.int32), bounds).astype(jnp.int32)
    ts = jnp.concatenate([ts, jnp.zeros((15,), jnp.int32)])

    q, k, v = _qkv_tc(feats, norm1_g, norm1_b, qkv_w, qkv_b)
    q = jnp.pad(q, ((0, NPAD - N), (0, 0))).reshape(NPAD * C)

    out_pad, _ = _sc_attention(i0s, i1, q, k, v, ts)
    attn_out = out_pad.reshape(NPAD, C)[:n]

    return _tail_tc(attn_out, feats, proj_w, proj_b, norm2_g, norm2_b,
                    fc1_w, fc1_b, fc2_w, fc2_b)
